# baseline (speedup 1.0000x reference)
import jax
import jax.numpy as jnp
from jax.experimental import pallas as pl
from jax.experimental.pallas import tpu as pltpu

_CHUNKS = 4


def _body(w_ref, out_ref):
    out_ref[...] = jnp.broadcast_to(w_ref[...][None], out_ref.shape)


def kernel(tokens, W_pos):
    B = tokens.shape[0]
    S = tokens.shape[1]
    D = W_pos.shape[1]
    rc = S // _CHUNKS
    return pl.pallas_call(
        _body,
        grid=(_CHUNKS,),
        in_specs=[pl.BlockSpec((rc, D), lambda i: (i, 0))],
        out_specs=pl.BlockSpec((B, rc, D), lambda i: (0, i, 0)),
        out_shape=jax.ShapeDtypeStruct((B, S, D), W_pos.dtype),
    )(W_pos[:S])


# speedup vs baseline: 1.1562x; 1.1562x over previous
import jax
import jax.numpy as jnp
from jax.experimental import pallas as pl
from jax.experimental.pallas import tpu as pltpu

_CHUNKS = 1


def kernel(tokens, W_pos):
    B = tokens.shape[0]
    S = tokens.shape[1]
    D = W_pos.shape[1]
    rc = S // _CHUNKS

    def body(w_hbm, out_hbm, vmem, in_sem, out_sem):
        in_copies = [
            pltpu.make_async_copy(
                w_hbm.at[pl.ds(i * rc, rc)], vmem.at[pl.ds(i * rc, rc)], in_sem
            )
            for i in range(_CHUNKS)
        ]
        in_copies[0].start()
        out_copies = []
        for i in range(_CHUNKS):
            in_copies[i].wait()
            if i + 1 < _CHUNKS:
                in_copies[i + 1].start()
            for b in range(B):
                c = pltpu.async_copy(
                    vmem.at[pl.ds(i * rc, rc)],
                    out_hbm.at[b, pl.ds(i * rc, rc)],
                    out_sem,
                )
                out_copies.append(c)
        for c in out_copies:
            c.wait()

    return pl.pallas_call(
        body,
        in_specs=[pl.BlockSpec(memory_space=pltpu.MemorySpace.HBM)],
        out_specs=pl.BlockSpec(memory_space=pltpu.MemorySpace.HBM),
        out_shape=jax.ShapeDtypeStruct((B, S, D), W_pos.dtype),
        scratch_shapes=[
            pltpu.VMEM((S, D), W_pos.dtype),
            pltpu.SemaphoreType.DMA,
            pltpu.SemaphoreType.DMA,
        ],
    )(W_pos[:S])
